# Initial kernel scaffold; baseline (speedup 1.0000x reference)
import jax
import jax.numpy as jnp
from jax.experimental import pallas as pl


def kernel(nodes, edges, receivers, senders, global_latent, node_graph_idx, edge_graph_idx, W, b):
    raise NotImplementedError("write your pallas kernel here")


# speedup vs baseline: 5.4620x; 5.4620x over previous
import functools

import jax
import jax.numpy as jnp
from jax import lax
from jax.experimental import pallas as pl
from jax.experimental.pallas import tpu as pltpu
from jax.experimental.pallas import tpu_sc as plsc

L = 16
G = 16
B = 4
N = 50000
D = 128
DL = D // L
SEGS = B * G
TOTAL = B * N
NC, NS = 2, 16
NW = NC * NS
RANGE = TOTAL // NW
CHUNK = 512
NWIN = (RANGE + 7 + CHUNK - 1) // CHUNK


def _mesh():
    return plsc.VectorSubcoreMesh(core_axis_name="c", subcore_axis_name="s")


@functools.partial(
    pl.kernel,
    out_type=(
        jax.ShapeDtypeStruct((NW, SEGS, L), jnp.float32),
        jax.ShapeDtypeStruct((NW, SEGS, L), jnp.float32),
    ),
    mesh=_mesh(),
    scratch_types=[
        pltpu.VMEM((CHUNK, D), jnp.float32),
        pltpu.VMEM((CHUNK + L,), jnp.int32),
        pltpu.VMEM((D,), jnp.float32),
        pltpu.VMEM((SEGS, L), jnp.float32),
        pltpu.VMEM((SEGS, L), jnp.float32),
    ],
)
def _partials(nodes, gidx, wt, part_out, cnt_out, nbuf, ibuf, wbuf, acc, cnt):
    w = lax.axis_index("s") * NC + lax.axis_index("c")
    start = w * RANGE
    end = start + RANGE
    a0 = (start // 8) * 8

    pltpu.sync_copy(wt, wbuf)
    zrow = jnp.zeros((L,), jnp.float32)
    ones = jnp.ones((L,), jnp.float32)
    for g in range(SEGS):
        acc[g] = zrow
        cnt[g] = zrow
    wv = [wbuf[pl.ds(d * L, L)] for d in range(DL)]

    def win_body(k, _):
        wk = a0 + k * CHUNK
        wkc = jnp.minimum(wk, TOTAL - CHUNK)
        pltpu.sync_copy(nodes.at[pl.ds(wkc, CHUNK)], nbuf)
        pltpu.sync_copy(gidx.at[pl.ds(wkc, CHUNK)], ibuf.at[pl.ds(0, CHUNK)])
        lo = jnp.maximum(start, wk) - wkc
        hi = jnp.minimum(end, wk + CHUNK) - wkc

        def node_body(n, _):
            g = ibuf[pl.ds(n, L)][0]
            t = nbuf[n, pl.ds(0, L)] * wv[0]
            for d in range(1, DL):
                t = t + nbuf[n, pl.ds(d * L, L)] * wv[d]
            plsc.addupdate(acc.at[g], t)
            plsc.addupdate(cnt.at[g], ones)
            return 0

        lax.fori_loop(lo, hi, node_body, 0)
        return 0

    lax.fori_loop(0, NWIN, win_body, 0)
    pltpu.sync_copy(acc, part_out.at[w])
    pltpu.sync_copy(cnt, cnt_out.at[w])


@functools.partial(
    pl.kernel,
    out_type=jax.ShapeDtypeStruct((SEGS, L), jnp.float32),
    mesh=_mesh(),
    scratch_types=[
        pltpu.VMEM((NW, 2, L), jnp.float32),
        pltpu.VMEM((NW, 2, L), jnp.float32),
        pltpu.VMEM((L,), jnp.float32),
        pltpu.VMEM((2 * L,), jnp.float32),
        pltpu.VMEM((2, L), jnp.float32),
    ],
)
def _finish(part, cnts, bt, out, pbuf, cbuf, bbuf, red, obuf):
    w = lax.axis_index("s") * NC + lax.axis_index("c")
    seg0 = w * 2
    pltpu.sync_copy(part.at[:, pl.ds(seg0, 2), :], pbuf)
    pltpu.sync_copy(cnts.at[:, pl.ds(seg0, 2), :], cbuf)
    pltpu.sync_copy(bt, bbuf)

    zrow = jnp.zeros((L,), jnp.float32)

    def pbody(p, carry):
        s0, c0, s1, c1 = carry
        return (s0 + pbuf[p, 0], c0 + cbuf[p, 0],
                s1 + pbuf[p, 1], c1 + cbuf[p, 1])

    s0, c0, s1, c1 = lax.fori_loop(0, NW, pbody, (zrow, zrow, zrow, zrow))

    bias = bbuf[...]
    red[pl.ds(L, L)] = zrow
    for i, (s, c) in enumerate(((s0, c0), (s1, c1))):
        red[pl.ds(0, L)] = s
        for sh in (8, 4, 2, 1):
            red[pl.ds(0, L)] = red[pl.ds(0, L)] + red[pl.ds(sh, L)]
        tot = red[pl.ds(0, L)]
        obuf[i] = tot / jnp.maximum(c, 1.0) + bias

    pltpu.sync_copy(obuf, out.at[pl.ds(seg0, 2)])


def kernel(nodes, edges, receivers, senders, global_latent, node_graph_idx,
           edge_graph_idx, W, b):
    flat_nodes = nodes.reshape(B * N, D)
    flat_idx = (node_graph_idx
                + (jnp.arange(B, dtype=jnp.int32) * G)[:, None]).reshape(-1)
    wt = W.reshape(D)
    bt = jnp.broadcast_to(b.astype(jnp.float32), (L,))
    part, cnt = _partials(flat_nodes, flat_idx, wt)
    res = _finish(part, cnt, bt)
    return res.reshape(B, G, L)[..., :1]


# speedup vs baseline: 7.8207x; 1.4318x over previous
import functools

import jax
import jax.numpy as jnp
from jax import lax
from jax.experimental import pallas as pl
from jax.experimental.pallas import tpu as pltpu
from jax.experimental.pallas import tpu_sc as plsc

L = 16
G = 16
B = 4
N = 50000
D = 128
DL = D // L
SEGS = B * G
TOTAL = B * N
NC, NS = 2, 16
NW = NC * NS
RANGE = TOTAL // NW
CHUNK = 416
NWIN = 16
GRP = 16


def _mesh():
    return plsc.VectorSubcoreMesh(core_axis_name="c", subcore_axis_name="s")


@functools.partial(
    pl.kernel,
    out_type=(
        jax.ShapeDtypeStruct((NW, SEGS, L), jnp.float32),
        jax.ShapeDtypeStruct((NW, SEGS, L), jnp.float32),
    ),
    mesh=_mesh(),
    scratch_types=[
        pltpu.VMEM((CHUNK, D), jnp.float32),
        pltpu.VMEM((CHUNK, D), jnp.float32),
        pltpu.VMEM((CHUNK + L,), jnp.int32),
        pltpu.VMEM((CHUNK + L,), jnp.int32),
        pltpu.VMEM((D,), jnp.float32),
        pltpu.VMEM((SEGS, L), jnp.float32),
        pltpu.VMEM((SEGS, L), jnp.float32),
        pltpu.SemaphoreType.DMA,
        pltpu.SemaphoreType.DMA,
        pltpu.SemaphoreType.DMA,
        pltpu.SemaphoreType.DMA,
    ],
)
def _partials(nodes, gidx, wt, part_out, cnt_out, nbuf0, nbuf1, ibuf0, ibuf1,
              wbuf, acc, cnt, *sems):
    nbufs, ibufs = (nbuf0, nbuf1), (ibuf0, ibuf1)
    w = lax.axis_index("s") * NC + lax.axis_index("c")
    start = w * RANGE
    end = start + RANGE
    a0 = (start // 8) * 8

    pltpu.sync_copy(wt, wbuf)
    zrow = jnp.zeros((L,), jnp.float32)
    ones = jnp.ones((L,), jnp.float32)
    for g in range(SEGS):
        acc[g] = zrow
        cnt[g] = zrow
    wv = [wbuf[pl.ds(d * L, L)] for d in range(DL)]

    def win_base(k):
        wk = a0 + k * CHUNK
        return wk, jnp.minimum(wk, TOTAL - CHUNK)

    def dma_start(k, par):
        _, wkc = win_base(k)
        pltpu.make_async_copy(nodes.at[pl.ds(wkc, CHUNK)], nbufs[par],
                              sems[2 * par]).start()
        pltpu.make_async_copy(gidx.at[pl.ds(wkc, CHUNK)],
                              ibufs[par].at[pl.ds(0, CHUNK)],
                              sems[2 * par + 1]).start()

    def dma_wait(par):
        pltpu.make_async_copy(nodes.at[pl.ds(0, CHUNK)], nbufs[par],
                              sems[2 * par]).wait()
        pltpu.make_async_copy(gidx.at[pl.ds(0, CHUNK)],
                              ibufs[par].at[pl.ds(0, CHUNK)],
                              sems[2 * par + 1]).wait()

    def process(k, par):
        wk, wkc = win_base(k)
        lo = jnp.maximum(start, wk) - wkc
        hi = jnp.minimum(end, wk + CHUNK) - wkc

        def accum_one(n, g):
            t = nbufs[par][n, pl.ds(0, L)] * wv[0]
            for d in range(1, DL):
                t = t + nbufs[par][n, pl.ds(d * L, L)] * wv[d]
            plsc.addupdate(acc.at[g], t)
            plsc.addupdate(cnt.at[g], ones)

        def grp_body(gi, _):
            base = lo + gi * GRP
            gv = ibufs[par][pl.ds(base, L)]
            for j in range(GRP):
                accum_one(base + j, gv[j])
            return 0

        ngrp = jnp.maximum(hi - lo, 0) // GRP
        lax.fori_loop(0, ngrp, grp_body, 0)

        def tail_body(n, _):
            accum_one(n, ibufs[par][pl.ds(n, L)][0])
            return 0

        lax.fori_loop(lo + ngrp * GRP, hi, tail_body, 0)

    dma_start(0, 0)

    def pair_body(p, _):
        for par in range(2):
            k = 2 * p + par

            @pl.when(k + 1 < NWIN)
            def _():
                dma_start(k + 1, 1 - par)

            dma_wait(par)
            process(k, par)
        return 0

    lax.fori_loop(0, NWIN // 2, pair_body, 0)
    pltpu.sync_copy(acc, part_out.at[w])
    pltpu.sync_copy(cnt, cnt_out.at[w])


@functools.partial(
    pl.kernel,
    out_type=jax.ShapeDtypeStruct((SEGS, L), jnp.float32),
    mesh=_mesh(),
    scratch_types=[
        pltpu.VMEM((NW, 2, L), jnp.float32),
        pltpu.VMEM((NW, 2, L), jnp.float32),
        pltpu.VMEM((L,), jnp.float32),
        pltpu.VMEM((2 * L,), jnp.float32),
        pltpu.VMEM((2, L), jnp.float32),
    ],
)
def _finish(part, cnts, bt, out, pbuf, cbuf, bbuf, red, obuf):
    w = lax.axis_index("s") * NC + lax.axis_index("c")
    seg0 = w * 2
    pltpu.sync_copy(part.at[:, pl.ds(seg0, 2), :], pbuf)
    pltpu.sync_copy(cnts.at[:, pl.ds(seg0, 2), :], cbuf)
    pltpu.sync_copy(bt, bbuf)

    zrow = jnp.zeros((L,), jnp.float32)

    def pbody(p, carry):
        s0, c0, s1, c1 = carry
        return (s0 + pbuf[p, 0], c0 + cbuf[p, 0],
                s1 + pbuf[p, 1], c1 + cbuf[p, 1])

    s0, c0, s1, c1 = lax.fori_loop(0, NW, pbody, (zrow, zrow, zrow, zrow))

    bias = bbuf[...]
    red[pl.ds(L, L)] = zrow
    for i, (s, c) in enumerate(((s0, c0), (s1, c1))):
        red[pl.ds(0, L)] = s
        for sh in (8, 4, 2, 1):
            red[pl.ds(0, L)] = red[pl.ds(0, L)] + red[pl.ds(sh, L)]
        tot = red[pl.ds(0, L)]
        obuf[i] = tot / jnp.maximum(c, 1.0) + bias

    pltpu.sync_copy(obuf, out.at[pl.ds(seg0, 2)])


def kernel(nodes, edges, receivers, senders, global_latent, node_graph_idx,
           edge_graph_idx, W, b):
    flat_nodes = nodes.reshape(B * N, D)
    flat_idx = (node_graph_idx
                + (jnp.arange(B, dtype=jnp.int32) * G)[:, None]).reshape(-1)
    wt = W.reshape(D)
    bt = jnp.broadcast_to(b.astype(jnp.float32), (L,))
    part, cnt = _partials(flat_nodes, flat_idx, wt)
    res = _finish(part, cnt, bt)
    return res.reshape(B, G, L)[..., :1]


# speedup vs baseline: 8.1168x; 1.0379x over previous
import functools

import jax
import jax.numpy as jnp
from jax import lax
from jax.experimental import pallas as pl
from jax.experimental.pallas import tpu as pltpu
from jax.experimental.pallas import tpu_sc as plsc

L = 16
G = 16
B = 4
N = 50000
D = 128
DL = D // L
SEGS = B * G
TOTAL = B * N
NC, NS = 2, 16
NW = NC * NS
RANGE = TOTAL // NW
CHUNK = 416
NWIN = 16
GRP = 16


def _mesh():
    return plsc.VectorSubcoreMesh(core_axis_name="c", subcore_axis_name="s")


@functools.partial(
    pl.kernel,
    out_type=(
        jax.ShapeDtypeStruct((NW, SEGS, L), jnp.float32),
        jax.ShapeDtypeStruct((NW, SEGS, L), jnp.float32),
    ),
    mesh=_mesh(),
    scratch_types=[
        pltpu.VMEM((CHUNK, D), jnp.float32),
        pltpu.VMEM((CHUNK, D), jnp.float32),
        pltpu.VMEM((CHUNK + L,), jnp.int32),
        pltpu.VMEM((CHUNK + L,), jnp.int32),
        pltpu.VMEM((D,), jnp.float32),
        pltpu.VMEM((SEGS, L), jnp.float32),
        pltpu.VMEM((SEGS, L), jnp.float32),
        pltpu.SemaphoreType.DMA,
        pltpu.SemaphoreType.DMA,
        pltpu.SemaphoreType.DMA,
        pltpu.SemaphoreType.DMA,
    ],
)
def _partials(nodes, gidx, wt, part_out, cnt_out, nbuf0, nbuf1, ibuf0, ibuf1,
              wbuf, acc, cnt, *sems):
    nbufs, ibufs = (nbuf0, nbuf1), (ibuf0, ibuf1)
    w = lax.axis_index("s") * NC + lax.axis_index("c")
    start = w * RANGE
    end = start + RANGE
    a0 = (start // 8) * 8

    pltpu.sync_copy(wt, wbuf)
    zrow = jnp.zeros((L,), jnp.float32)
    ones = jnp.ones((L,), jnp.float32)
    for g in range(SEGS):
        acc[g] = zrow
        cnt[g] = zrow
    wv = [wbuf[pl.ds(d * L, L)] for d in range(DL)]

    def win_base(k):
        wk = a0 + k * CHUNK
        return wk, jnp.minimum(wk, TOTAL - CHUNK)

    def dma_start(k, par):
        _, wkc = win_base(k)
        pltpu.make_async_copy(nodes.at[pl.ds(wkc, CHUNK)], nbufs[par],
                              sems[2 * par]).start()
        pltpu.make_async_copy(gidx.at[pl.ds(wkc, CHUNK)],
                              ibufs[par].at[pl.ds(0, CHUNK)],
                              sems[2 * par + 1]).start()

    def dma_wait(par):
        pltpu.make_async_copy(nodes.at[pl.ds(0, CHUNK)], nbufs[par],
                              sems[2 * par]).wait()
        pltpu.make_async_copy(gidx.at[pl.ds(0, CHUNK)],
                              ibufs[par].at[pl.ds(0, CHUNK)],
                              sems[2 * par + 1]).wait()

    def process(k, par):
        wk, wkc = win_base(k)
        lo = jnp.maximum(start, wk) - wkc
        hi = jnp.minimum(end, wk + CHUNK) - wkc

        def accum_one(n, g):
            prods = [nbufs[par][n, pl.ds(d * L, L)] * wv[d] for d in range(DL)]
            while len(prods) > 1:
                prods = [prods[i] + prods[i + 1]
                         for i in range(0, len(prods) - 1, 2)] + prods[len(prods) & ~1:]
            plsc.addupdate(acc.at[g], prods[0])
            plsc.addupdate(cnt.at[g], ones)

        def grp_body(gi, _):
            base = lo + gi * GRP
            gv = ibufs[par][pl.ds(base, L)]
            for j in range(GRP):
                accum_one(base + j, gv[j])
            return 0

        ngrp = jnp.maximum(hi - lo, 0) // GRP
        lax.fori_loop(0, ngrp, grp_body, 0)

        def tail_body(n, _):
            accum_one(n, ibufs[par][pl.ds(n, L)][0])
            return 0

        lax.fori_loop(lo + ngrp * GRP, hi, tail_body, 0)

    dma_start(0, 0)

    def pair_body(p, _):
        for par in range(2):
            k = 2 * p + par

            @pl.when(k + 1 < NWIN)
            def _():
                dma_start(k + 1, 1 - par)

            dma_wait(par)
            process(k, par)
        return 0

    lax.fori_loop(0, NWIN // 2, pair_body, 0)
    pltpu.sync_copy(acc, part_out.at[w])
    pltpu.sync_copy(cnt, cnt_out.at[w])


@functools.partial(
    pl.kernel,
    out_type=jax.ShapeDtypeStruct((SEGS, L), jnp.float32),
    mesh=_mesh(),
    scratch_types=[
        pltpu.VMEM((NW, 2, L), jnp.float32),
        pltpu.VMEM((NW, 2, L), jnp.float32),
        pltpu.VMEM((L,), jnp.float32),
        pltpu.VMEM((2 * L,), jnp.float32),
        pltpu.VMEM((2, L), jnp.float32),
    ],
)
def _finish(part, cnts, bt, out, pbuf, cbuf, bbuf, red, obuf):
    w = lax.axis_index("s") * NC + lax.axis_index("c")
    seg0 = w * 2
    pltpu.sync_copy(part.at[:, pl.ds(seg0, 2), :], pbuf)
    pltpu.sync_copy(cnts.at[:, pl.ds(seg0, 2), :], cbuf)
    pltpu.sync_copy(bt, bbuf)

    zrow = jnp.zeros((L,), jnp.float32)

    def pbody(p, carry):
        s0, c0, s1, c1 = carry
        return (s0 + pbuf[p, 0], c0 + cbuf[p, 0],
                s1 + pbuf[p, 1], c1 + cbuf[p, 1])

    s0, c0, s1, c1 = lax.fori_loop(0, NW, pbody, (zrow, zrow, zrow, zrow))

    bias = bbuf[...]
    red[pl.ds(L, L)] = zrow
    for i, (s, c) in enumerate(((s0, c0), (s1, c1))):
        red[pl.ds(0, L)] = s
        for sh in (8, 4, 2, 1):
            red[pl.ds(0, L)] = red[pl.ds(0, L)] + red[pl.ds(sh, L)]
        tot = red[pl.ds(0, L)]
        obuf[i] = tot / jnp.maximum(c, 1.0) + bias

    pltpu.sync_copy(obuf, out.at[pl.ds(seg0, 2)])


def kernel(nodes, edges, receivers, senders, global_latent, node_graph_idx,
           edge_graph_idx, W, b):
    flat_nodes = nodes.reshape(B * N, D)
    flat_idx = (node_graph_idx
                + (jnp.arange(B, dtype=jnp.int32) * G)[:, None]).reshape(-1)
    wt = W.reshape(D)
    bt = jnp.broadcast_to(b.astype(jnp.float32), (L,))
    part, cnt = _partials(flat_nodes, flat_idx, wt)
    res = _finish(part, cnt, bt)
    return res.reshape(B, G, L)[..., :1]


# speedup vs baseline: 11.8033x; 1.4542x over previous
import functools

import jax
import jax.numpy as jnp
from jax import lax
from jax.experimental import pallas as pl
from jax.experimental.pallas import tpu as pltpu
from jax.experimental.pallas import tpu_sc as plsc

L = 16
G = 16
B = 4
N = 50000
D = 128
DL = D // L
SEGS = B * G
TOTAL = B * N
NC, NS = 2, 16
NW = NC * NS
RANGE = TOTAL // NW
CHUNK = 416
NWIN = 16
GRP = 16


def _mesh():
    return plsc.VectorSubcoreMesh(core_axis_name="c", subcore_axis_name="s")


@functools.partial(
    pl.kernel,
    out_type=(
        jax.ShapeDtypeStruct((NW, SEGS, L), jnp.float32),
        jax.ShapeDtypeStruct((NW, SEGS, L), jnp.float32),
    ),
    mesh=_mesh(),
    scratch_types=[
        pltpu.VMEM((CHUNK, D), jnp.float32),
        pltpu.VMEM((CHUNK, D), jnp.float32),
        pltpu.VMEM((CHUNK + L,), jnp.int32),
        pltpu.VMEM((CHUNK + L,), jnp.int32),
        pltpu.VMEM((D,), jnp.float32),
        pltpu.VMEM((SEGS, L), jnp.float32),
        pltpu.VMEM((SEGS, L), jnp.float32),
        pltpu.SemaphoreType.DMA,
        pltpu.SemaphoreType.DMA,
        pltpu.SemaphoreType.DMA,
        pltpu.SemaphoreType.DMA,
    ],
)
def _partials(nodes, gidx, wt, part_out, cnt_out, nbuf0, nbuf1, ibuf0, ibuf1,
              wbuf, acc, cnt, *sems):
    nbufs, ibufs = (nbuf0, nbuf1), (ibuf0, ibuf1)
    w = lax.axis_index("s") * NC + lax.axis_index("c")
    start = w * RANGE
    end = start + RANGE
    a0 = (start // 8) * 8

    pltpu.sync_copy(wt, wbuf)
    zrow = jnp.zeros((L,), jnp.float32)
    ones = jnp.ones((L,), jnp.float32)
    for g in range(SEGS):
        acc[g] = zrow
        cnt[g] = zrow
    wv = [wbuf[pl.ds(d * L, L)] for d in range(DL)]

    def win_base(k):
        wk = a0 + k * CHUNK
        return wk, jnp.minimum(wk, TOTAL - CHUNK)

    def dma_start(k, par):
        _, wkc = win_base(k)
        pltpu.make_async_copy(nodes.at[pl.ds(wkc, CHUNK)], nbufs[par],
                              sems[2 * par]).start()
        pltpu.make_async_copy(gidx.at[pl.ds(wkc, CHUNK)],
                              ibufs[par].at[pl.ds(0, CHUNK)],
                              sems[2 * par + 1]).start()

    def dma_wait(par):
        pltpu.make_async_copy(nodes.at[pl.ds(0, CHUNK)], nbufs[par],
                              sems[2 * par]).wait()
        pltpu.make_async_copy(gidx.at[pl.ds(0, CHUNK)],
                              ibufs[par].at[pl.ds(0, CHUNK)],
                              sems[2 * par + 1]).wait()

    def process(k, par):
        wk, wkc = win_base(k)
        lo = jnp.maximum(start, wk) - wkc
        hi = jnp.minimum(end, wk + CHUNK) - wkc

        def accum_one(n, g):
            prods = [nbufs[par][n, pl.ds(d * L, L)] * wv[d] for d in range(DL)]
            while len(prods) > 1:
                prods = [prods[i] + prods[i + 1]
                         for i in range(0, len(prods) - 1, 2)] + prods[len(prods) & ~1:]
            plsc.addupdate(acc.at[g], prods[0])
            plsc.addupdate(cnt.at[g], ones)

        ngrp = jnp.maximum(hi - lo, 0) // GRP

        @plsc.parallel_loop(0, ngrp)
        def grp_body(gi):
            base = lo + gi * GRP
            gv = ibufs[par][pl.ds(base, L)]
            for j in range(GRP):
                accum_one(base + j, gv[j])

        def tail_body(n, _):
            accum_one(n, ibufs[par][pl.ds(n, L)][0])
            return 0

        lax.fori_loop(lo + ngrp * GRP, hi, tail_body, 0)

    dma_start(0, 0)

    def pair_body(p, _):
        for par in range(2):
            k = 2 * p + par

            @pl.when(k + 1 < NWIN)
            def _():
                dma_start(k + 1, 1 - par)

            dma_wait(par)
            process(k, par)
        return 0

    lax.fori_loop(0, NWIN // 2, pair_body, 0)
    pltpu.sync_copy(acc, part_out.at[w])
    pltpu.sync_copy(cnt, cnt_out.at[w])


@functools.partial(
    pl.kernel,
    out_type=jax.ShapeDtypeStruct((SEGS, L), jnp.float32),
    mesh=_mesh(),
    scratch_types=[
        pltpu.VMEM((NW, 2, L), jnp.float32),
        pltpu.VMEM((NW, 2, L), jnp.float32),
        pltpu.VMEM((L,), jnp.float32),
        pltpu.VMEM((2 * L,), jnp.float32),
        pltpu.VMEM((2, L), jnp.float32),
    ],
)
def _finish(part, cnts, bt, out, pbuf, cbuf, bbuf, red, obuf):
    w = lax.axis_index("s") * NC + lax.axis_index("c")
    seg0 = w * 2
    pltpu.sync_copy(part.at[:, pl.ds(seg0, 2), :], pbuf)
    pltpu.sync_copy(cnts.at[:, pl.ds(seg0, 2), :], cbuf)
    pltpu.sync_copy(bt, bbuf)

    zrow = jnp.zeros((L,), jnp.float32)

    def pbody(p, carry):
        s0, c0, s1, c1 = carry
        return (s0 + pbuf[p, 0], c0 + cbuf[p, 0],
                s1 + pbuf[p, 1], c1 + cbuf[p, 1])

    s0, c0, s1, c1 = lax.fori_loop(0, NW, pbody, (zrow, zrow, zrow, zrow))

    bias = bbuf[...]
    red[pl.ds(L, L)] = zrow
    for i, (s, c) in enumerate(((s0, c0), (s1, c1))):
        red[pl.ds(0, L)] = s
        for sh in (8, 4, 2, 1):
            red[pl.ds(0, L)] = red[pl.ds(0, L)] + red[pl.ds(sh, L)]
        tot = red[pl.ds(0, L)]
        obuf[i] = tot / jnp.maximum(c, 1.0) + bias

    pltpu.sync_copy(obuf, out.at[pl.ds(seg0, 2)])


def kernel(nodes, edges, receivers, senders, global_latent, node_graph_idx,
           edge_graph_idx, W, b):
    flat_nodes = nodes.reshape(B * N, D)
    flat_idx = (node_graph_idx
                + (jnp.arange(B, dtype=jnp.int32) * G)[:, None]).reshape(-1)
    wt = W.reshape(D)
    bt = jnp.broadcast_to(b.astype(jnp.float32), (L,))
    part, cnt = _partials(flat_nodes, flat_idx, wt)
    res = _finish(part, cnt, bt)
    return res.reshape(B, G, L)[..., :1]
